# baseline (speedup 1.0000x reference)
import functools

import jax
import jax.numpy as jnp
from jax import lax
from jax.experimental import pallas as pl
from jax.experimental.pallas import tpu as pltpu
from jax.experimental.pallas import tpu_sc as plsc

VOCAB = 1000000
EMBED = 64
BATCH = 16384

NUM_CORES = 2
NUM_SUBCORES = 16
LANES = 16
NW = NUM_CORES * NUM_SUBCORES
B_PER_W = BATCH // NW
GROUPS = B_PER_W // LANES


def _make_sc_kernel():
    mesh = plsc.VectorSubcoreMesh(core_axis_name="c", subcore_axis_name="s")

    @functools.partial(
        pl.kernel,
        mesh=mesh,
        out_type=jax.ShapeDtypeStruct((BATCH,), jnp.float32),
        compiler_params=pltpu.CompilerParams(
            needs_layout_passes=False, use_tc_tiling_on_sc=False),
        scratch_types=[
            pltpu.VMEM((B_PER_W,), jnp.int32),
            pltpu.VMEM((B_PER_W,), jnp.int32),
            pltpu.VMEM((B_PER_W, EMBED), jnp.float32),
            pltpu.VMEM((B_PER_W, EMBED), jnp.float32),
            pltpu.VMEM((B_PER_W,), jnp.float32),
            pltpu.SemaphoreType.DMA,
            pltpu.SemaphoreType.DMA,
        ],
    )
    def k(idx0_hbm, idx1_hbm, emb1_hbm, emb2_hbm, out_hbm,
          idx0_v, idx1_v, u_v, v_v, out_v, sem0, sem1):
        wid = lax.axis_index("s") * NUM_CORES + lax.axis_index("c")
        base = wid * B_PER_W

        pltpu.sync_copy(idx0_hbm.at[pl.ds(base, B_PER_W)], idx0_v)
        pltpu.sync_copy(idx1_hbm.at[pl.ds(base, B_PER_W)], idx1_v)
        cp0 = pltpu.async_copy(emb1_hbm.at[idx0_v], u_v, sem0)
        cp1 = pltpu.async_copy(emb2_hbm.at[idx1_v], v_v, sem1)
        cp0.wait()
        cp1.wait()

        lane = lax.iota(jnp.int32, LANES)
        cols = [jnp.full((LANES,), d, jnp.int32) for d in range(EMBED)]

        def group(g, carry):
            rows = g * LANES + lane
            acc = jnp.zeros((LANES,), jnp.float32)
            for d in range(EMBED):
                u = plsc.load_gather(u_v, [rows, cols[d]])
                v = plsc.load_gather(v_v, [rows, cols[d]])
                acc = acc + u * v
            out_v[pl.ds(g * LANES, LANES)] = 1.0 / (1.0 + jnp.exp(-acc))
            return carry

        lax.fori_loop(0, GROUPS, group, 0)
        pltpu.sync_copy(out_v, out_hbm.at[pl.ds(base, B_PER_W)])

    return k


_sc_kernel = _make_sc_kernel()


@jax.jit
def kernel(X_batch, emb1, emb2):
    idx0 = X_batch[:, 0].astype(jnp.int32)
    idx1 = X_batch[:, 1].astype(jnp.int32)
    return _sc_kernel(idx0, idx1, emb1, emb2)
